# baseline (speedup 1.0000x reference)
import functools
import jax
import jax.numpy as jnp
from jax import lax
from jax.experimental import pallas as pl
from jax.experimental.pallas import tpu as pltpu
from jax.experimental.pallas import tpu_sc as plsc

NC = 2
NS = 16
NW = NC * NS
LANES = 16
BATCH = 16384
EMB = 64
BPW = BATCH // NW
CHUNK = 128
NCHUNK = BPW // CHUNK


def _body(user_hbm, movie_hbm, uemb_hbm, memb_hbm, ubias_hbm, mbias_hbm,
          gbias_hbm, out_hbm,
          uidx_v, midx_v, urows_v, mrows_v, ubias_v, mbias_v, gb_v, out_v,
          buf_v, sem):
    wid = lax.axis_index("s") * NC + lax.axis_index("c")
    base = wid * BPW

    for j in range(NCHUNK):
        pltpu.sync_copy(user_hbm.at[pl.ds(base + j * CHUNK, CHUNK)],
                        uidx_v.at[j])
        pltpu.sync_copy(movie_hbm.at[pl.ds(base + j * CHUNK, CHUNK)],
                        midx_v.at[j])
    pltpu.sync_copy(gbias_hbm, gb_v.at[pl.ds(0, 1)])

    copies = []
    for j in range(NCHUNK):
        sl = pl.ds(j * CHUNK, CHUNK)
        copies.append(pltpu.async_copy(
            uemb_hbm.at[uidx_v.at[j]], urows_v.at[sl], sem))
        copies.append(pltpu.async_copy(
            memb_hbm.at[midx_v.at[j]], mrows_v.at[sl], sem))
        copies.append(pltpu.async_copy(
            ubias_hbm.at[uidx_v.at[j]], ubias_v.at[sl], sem))
        copies.append(pltpu.async_copy(
            mbias_hbm.at[midx_v.at[j]], mbias_v.at[sl], sem))
    for c in copies:
        c.wait()

    gb = gb_v[pl.ds(0, LANES)][0]
    base_idx = jnp.arange(LANES, dtype=jnp.int32) * (LANES + 1)

    def group(g, carry):
        rbase = g * LANES
        for r in range(LANES):
            row = rbase + r
            acc = None
            for d in range(0, EMB, LANES):
                u = urows_v[row, pl.ds(d, LANES)]
                m = mrows_v[row, pl.ds(d, LANES)]
                p = u * m
                acc = p if acc is None else acc + p
            buf_v[pl.ds(r * (LANES + 1), LANES)] = acc
        tot = None
        for c in range(LANES):
            col = plsc.load_gather(buf_v, [base_idx + c])
            tot = col if tot is None else tot + col
        vec = (tot + ubias_v[pl.ds(rbase, LANES)]
               + mbias_v[pl.ds(rbase, LANES)] + gb)
        out_v[pl.ds(rbase, LANES)] = vec
        return carry

    lax.fori_loop(0, BPW // LANES, group, 0)

    pltpu.sync_copy(out_v, out_hbm.at[pl.ds(base, BPW)])


def kernel(user, movie, user_embedding, movie_embedding,
           user_bias_embedding, movie_bias_embedding, global_bias):
    mesh = plsc.VectorSubcoreMesh(core_axis_name="c", subcore_axis_name="s",
                                  num_cores=NC, num_subcores=NS)
    ubias = user_bias_embedding.reshape(-1)
    mbias = movie_bias_embedding.reshape(-1)
    run = pl.kernel(
        _body,
        out_type=jax.ShapeDtypeStruct((BATCH,), jnp.float32),
        mesh=mesh,
        compiler_params=pltpu.CompilerParams(needs_layout_passes=False,
                                             use_tc_tiling_on_sc=False),
        scratch_types=[
            pltpu.VMEM((NCHUNK, CHUNK), jnp.int32),
            pltpu.VMEM((NCHUNK, CHUNK), jnp.int32),
            pltpu.VMEM((BPW, EMB), jnp.float32),
            pltpu.VMEM((BPW, EMB), jnp.float32),
            pltpu.VMEM((BPW,), jnp.float32),
            pltpu.VMEM((BPW,), jnp.float32),
            pltpu.VMEM((LANES,), jnp.float32),
            pltpu.VMEM((BPW,), jnp.float32),
            pltpu.VMEM((LANES * (LANES + 1),), jnp.float32),
            pltpu.SemaphoreType.DMA,
        ],
    )
    return run(user, movie, user_embedding, movie_embedding, ubias, mbias,
               global_bias)
